# Initial kernel scaffold; baseline (speedup 1.0000x reference)
import functools

import jax
import jax.numpy as jnp
from jax import lax
from jax.experimental import pallas as pl
from jax.experimental.pallas import tpu as pltpu
from jax.experimental.pallas import tpu_sc as plsc

_VOCAB1 = 1000001
_WORD_DIM = 32
_B = 16384 * 20

_INFO = plsc.get_sparse_core_info()
_NW = _INFO.num_cores * _INFO.num_subcores
_B_PER_W = _B // _NW
_CHUNK = 2048
_NCHUNK = _B_PER_W // _CHUNK


def _gather_body(table_hbm, idx_hbm, out_hbm, idx_v, rows_v, sem):
    wid = lax.axis_index("s") * _INFO.num_cores + lax.axis_index("c")
    base = wid * _B_PER_W
    pltpu.sync_copy(idx_hbm.at[pl.ds(base, _B_PER_W)], idx_v)
    for j in range(_NCHUNK):
        pltpu.async_copy(
            table_hbm.at[idx_v.at[pl.ds(j * _CHUNK, _CHUNK)]],
            rows_v,
            sem,
        ).wait()
        pltpu.sync_copy(
            rows_v, out_hbm.at[pl.ds(base + j * _CHUNK, _CHUNK)]
        )


@jax.jit
def _gather(table, idx_flat):
    mesh = plsc.VectorSubcoreMesh(core_axis_name="c", subcore_axis_name="s")
    k = pl.kernel(
        _gather_body,
        out_type=jax.ShapeDtypeStruct((_B, _WORD_DIM), jnp.float32),
        mesh=mesh,
        scratch_types=[
            pltpu.VMEM((_B_PER_W,), jnp.int32),
            pltpu.VMEM((_CHUNK, _WORD_DIM), jnp.float32),
            pltpu.SemaphoreType.DMA,
        ],
    )
    return k(table, idx_flat)


def kernel(inputs, embeddings):
    batch, hist = inputs.shape
    out = _gather(embeddings, inputs.reshape(-1))
    return out.reshape(batch, hist, _WORD_DIM)


# speedup vs baseline: 1.5097x; 1.5097x over previous
import functools

import jax
import jax.numpy as jnp
from jax import lax
from jax.experimental import pallas as pl
from jax.experimental.pallas import tpu as pltpu
from jax.experimental.pallas import tpu_sc as plsc

_VOCAB1 = 1000001
_WORD_DIM = 32
_B = 16384 * 20

_INFO = plsc.get_sparse_core_info()
_NW = _INFO.num_cores * _INFO.num_subcores
_B_PER_W = _B // _NW
_CHUNK = 2048
_NCHUNK = _B_PER_W // _CHUNK


def _gather_body(table_hbm, idx_hbm, out_hbm, idx_v, rows_v, sem):
    wid = lax.axis_index("s") * _INFO.num_cores + lax.axis_index("c")
    base = wid * _B_PER_W
    pltpu.sync_copy(idx_hbm.at[pl.ds(base, _B_PER_W)], idx_v)
    for j in range(_NCHUNK):
        pltpu.async_copy(
            table_hbm.at[idx_v.at[pl.ds(j * _CHUNK, _CHUNK)]],
            rows_v,
            sem,
        ).wait()
        pltpu.sync_copy(
            rows_v, out_hbm.at[pl.ds(base + j * _CHUNK, _CHUNK)]
        )


@jax.jit
def _gather(table, idx_flat):
    mesh = plsc.VectorSubcoreMesh(core_axis_name="c", subcore_axis_name="s")
    k = pl.kernel(
        _gather_body,
        out_type=jax.ShapeDtypeStruct((_B, _WORD_DIM), jnp.float32),
        mesh=mesh,
        scratch_types=[
            pltpu.VMEM((_B_PER_W,), jnp.int32),
            pltpu.VMEM((_CHUNK, _WORD_DIM), jnp.float32),
            pltpu.SemaphoreType.DMA,
        ],
        compiler_params=pltpu.CompilerParams(use_tc_tiling_on_sc=False),
    )
    return k(table, idx_flat)


def kernel(inputs, embeddings):
    batch, hist = inputs.shape
    out = _gather(embeddings, inputs.reshape(-1))
    return out.reshape(batch, hist, _WORD_DIM)
